# baseline (speedup 1.0000x reference)
import jax
import jax.numpy as jnp
from jax import lax
from jax.experimental import pallas as pl
from jax.experimental.pallas import tpu as pltpu
from jax.experimental.pallas import tpu_sc as plsc

VOCAB = 1_000_000
D = 32
SEQ = 200
BATCH = 4096

NC, NS, L = 2, 16, 16
NW = NC * NS
BB = 128
LCH = 8


def _tpe_body(table, tokens_t, pos_r, out_hbm,
              idx_v, grows_v, tbuf_v, pos_v,
              isem, gsem0, gsem1, osem0, osem1):
    wid = lax.axis_index("s") * NC + lax.axis_index("c")
    b0 = wid * BB

    pltpu.sync_copy(pos_r, pos_v)
    pltpu.sync_copy(tokens_t.at[pl.ds(0, LCH), pl.ds(b0, BB)], idx_v.at[0])

    gsems = (gsem0, gsem1)
    osems = (osem0, osem1)

    def fire(l, buf):
        c = lax.rem(lax.div(l, LCH), 2)
        li = lax.rem(l, LCH)
        pltpu.async_copy(
            table.at[idx_v.at[c, li]], grows_v.at[buf], gsems[buf]
        )

    def wait_gather(l, buf):
        c = lax.rem(lax.div(l, LCH), 2)
        li = lax.rem(l, LCH)
        pltpu.make_async_copy(
            table.at[idx_v.at[c, li]], grows_v.at[buf], gsems[buf]
        ).wait()

    def fire_out(l, buf):
        for h in range(D // 8):
            pltpu.async_copy(
                tbuf_v.at[buf, pl.ds(h * 8 * BB, 8 * BB)],
                out_hbm.at[l, h, wid],
                osems[buf],
            )

    def wait_out(l, buf):
        for h in range(D // 8):
            pltpu.make_async_copy(
                tbuf_v.at[buf, pl.ds(h * 8 * BB, 8 * BB)],
                out_hbm.at[l, h, wid],
                osems[buf],
            ).wait()

    lane = lax.iota(jnp.int32, L)
    base0 = lane * BB
    base1 = (lane + L) * BB

    def process(l, buf):
        p0 = pos_v[l, pl.ds(0, L)]
        p1 = pos_v[l, pl.ds(L, L)]
        for t in range(BB):
            v0 = grows_v[buf, t, pl.ds(0, L)] + p0
            plsc.store_scatter(tbuf_v.at[buf], [base0 + t], v0)
            v1 = grows_v[buf, t, pl.ds(L, L)] + p1
            plsc.store_scatter(tbuf_v.at[buf], [base1 + t], v1)

    fire(0, 0)

    @pl.loop(0, SEQ, step=2)
    def _l(l):
        fire(l + 1, 1)
        wait_gather(l, 0)

        @pl.when(l >= 2)
        def _w0():
            wait_out(l - 2, 0)

        process(l, 0)
        fire_out(l, 0)

        @pl.when(lax.rem(l + 2, LCH) == 0)
        def _nx():
            c = lax.rem(lax.div(l + 2, LCH), 2)
            pltpu.sync_copy(
                tokens_t.at[pl.ds(l + 2, LCH), pl.ds(b0, BB)], idx_v.at[c]
            )

        @pl.when(l + 2 < SEQ)
        def _pf():
            fire(l + 2, 0)

        wait_gather(l + 1, 1)

        @pl.when(l >= 2)
        def _w1():
            wait_out(l - 1, 1)

        process(l + 1, 1)
        fire_out(l + 1, 1)

    wait_out(SEQ - 2, 0)
    wait_out(SEQ - 1, 1)


def kernel(tokens, token_table, position_table):
    mesh = plsc.VectorSubcoreMesh(core_axis_name="c", subcore_axis_name="s")
    run = pl.kernel(
        _tpe_body,
        out_type=jax.ShapeDtypeStruct((SEQ, D // 8, NW, 8 * BB), jnp.float32),
        mesh=mesh,
        scratch_types=[
            pltpu.VMEM((2, LCH, BB), jnp.int32),
            pltpu.VMEM((2, BB, D), jnp.float32),
            pltpu.VMEM((2, D * BB), jnp.float32),
            pltpu.VMEM((SEQ, D), jnp.float32),
            pltpu.SemaphoreType.DMA,
            pltpu.SemaphoreType.DMA,
            pltpu.SemaphoreType.DMA,
            pltpu.SemaphoreType.DMA,
            pltpu.SemaphoreType.DMA,
        ],
        compiler_params=pltpu.CompilerParams(
            use_tc_tiling_on_sc=False, needs_layout_passes=False
        ),
    )
    tokens_t = tokens.T.astype(jnp.int32)
    raw = run(token_table, tokens_t, position_table)
    raw = raw.reshape(SEQ, D // 8, NW, 8, BB)
    return raw.transpose(2, 4, 0, 1, 3).reshape(BATCH, SEQ, D)


# speedup vs baseline: 1.1383x; 1.1383x over previous
import jax
import jax.numpy as jnp
from jax import lax
from jax.experimental import pallas as pl
from jax.experimental.pallas import tpu as pltpu
from jax.experimental.pallas import tpu_sc as plsc

VOCAB = 1_000_000
D = 32
SEQ = 200
BATCH = 4096

NC, NS, L = 2, 16, 16
NW = NC * NS
BB = 128
LCH = 8
NB = 4
TG = 8
SCW = 3976


def _tpe_body(table, tokens_t, pos_r, out_hbm,
              idx_v, grows_v, tbuf_v, pos_v,
              isem, gsem0, gsem1, gsem2, gsem3,
              osem0, osem1, osem2, osem3):
    wid = lax.axis_index("s") * NC + lax.axis_index("c")
    b0 = wid * BB

    pltpu.sync_copy(pos_r, pos_v)
    pltpu.sync_copy(tokens_t.at[pl.ds(0, LCH), pl.ds(b0, BB)], idx_v.at[0])

    gsems = (gsem0, gsem1, gsem2, gsem3)
    osems = (osem0, osem1, osem2, osem3)

    def fire(l, buf):
        c = lax.rem(lax.div(l, LCH), 2)
        li = lax.rem(l, LCH)
        pltpu.async_copy(
            table.at[idx_v.at[c, li]], grows_v.at[buf], gsems[buf]
        )

    def wait_gather(l, buf):
        c = lax.rem(lax.div(l, LCH), 2)
        li = lax.rem(l, LCH)
        pltpu.make_async_copy(
            table.at[idx_v.at[c, li]], grows_v.at[buf], gsems[buf]
        ).wait()

    def fire_out(l, buf):
        for h in range(D // 8):
            pltpu.async_copy(
                tbuf_v.at[buf, pl.ds(h * 8 * BB, 8 * BB)],
                out_hbm.at[l, h, wid],
                osems[buf],
            )

    def wait_out(l, buf):
        for h in range(D // 8):
            pltpu.make_async_copy(
                tbuf_v.at[buf, pl.ds(h * 8 * BB, 8 * BB)],
                out_hbm.at[l, h, wid],
                osems[buf],
            ).wait()

    lane = lax.iota(jnp.int32, L)
    idx0 = [lane * BB + k for k in range(TG)]
    idx1 = [(lane + L) * BB + k for k in range(TG)]

    def process(l, buf):
        p0 = pos_v[l, pl.ds(0, L)]
        p1 = pos_v[l, pl.ds(L, L)]
        for g in range(BB // TG):
            win = tbuf_v.at[buf, pl.ds(g * TG, SCW)]
            vs = []
            for k in range(TG):
                t = g * TG + k
                vs.append(grows_v[buf, t, pl.ds(0, L)] + p0)
                vs.append(grows_v[buf, t, pl.ds(L, L)] + p1)
            for k in range(TG):
                plsc.store_scatter(win, [idx0[k]], vs[2 * k])
                plsc.store_scatter(win, [idx1[k]], vs[2 * k + 1])

    for b in range(NB - 1):
        fire(b, b)

    @pl.loop(0, SEQ, step=NB)
    def _l(l):
        for b in range(NB):
            s = l + b
            sp = s + NB - 1
            pb = (b + NB - 1) % NB

            @pl.when(
                jnp.logical_and(lax.rem(sp, LCH) == 0, sp < SEQ)
            )
            def _chunk():
                c = lax.rem(lax.div(sp, LCH), 2)
                pltpu.sync_copy(
                    tokens_t.at[pl.ds(sp, LCH), pl.ds(b0, BB)], idx_v.at[c]
                )

            @pl.when(sp < SEQ)
            def _pf():
                fire(sp, pb)

            wait_gather(s, b)

            @pl.when(s >= NB)
            def _wo():
                wait_out(s - NB, b)

            process(s, b)
            fire_out(s, b)

    for b in range(NB):
        wait_out(SEQ - NB + b, b)


def kernel(tokens, token_table, position_table):
    mesh = plsc.VectorSubcoreMesh(core_axis_name="c", subcore_axis_name="s")
    run = pl.kernel(
        _tpe_body,
        out_type=jax.ShapeDtypeStruct((SEQ, D // 8, NW, 8 * BB), jnp.float32),
        mesh=mesh,
        scratch_types=[
            pltpu.VMEM((2, LCH, BB), jnp.int32),
            pltpu.VMEM((NB, BB, D), jnp.float32),
            pltpu.VMEM((NB, D * BB + 8), jnp.float32),
            pltpu.VMEM((SEQ, D), jnp.float32),
            pltpu.SemaphoreType.DMA,
            pltpu.SemaphoreType.DMA,
            pltpu.SemaphoreType.DMA,
            pltpu.SemaphoreType.DMA,
            pltpu.SemaphoreType.DMA,
            pltpu.SemaphoreType.DMA,
            pltpu.SemaphoreType.DMA,
            pltpu.SemaphoreType.DMA,
            pltpu.SemaphoreType.DMA,
        ],
        compiler_params=pltpu.CompilerParams(
            use_tc_tiling_on_sc=False, needs_layout_passes=False
        ),
    )
    tokens_t = tokens.T.astype(jnp.int32)
    raw = run(token_table, tokens_t, position_table)
    raw = raw.reshape(SEQ, D // 8, NW, 8, BB)
    return raw.transpose(2, 4, 0, 1, 3).reshape(BATCH, SEQ, D)
